# baseline (speedup 1.0000x reference)
import functools

import jax
import jax.numpy as jnp
from jax import lax
from jax.experimental import pallas as pl
from jax.experimental.pallas import tpu as pltpu
from jax.experimental.pallas import tpu_sc as plsc


def kernel(ctx, class_embeds, cls_embed, sep_embed, class_lens):
    n_cls, W, d = class_embeds.shape
    n_ctx = ctx.shape[0]
    hdr = 1 + n_ctx
    max_len = 2 + n_ctx + W
    T = W + 1

    NC, NS = 2, 16
    NW = NC * NS
    CPW = -(-n_cls // NW)
    rem = n_cls - (NW - 1) * CPW
    G = CPW // W
    MW = 128
    MPAD = -(-n_cls // MW) * MW
    mrem = n_cls % MW or MW

    mesh = plsc.VectorSubcoreMesh(
        core_axis_name="c", subcore_axis_name="s",
        num_cores=NC, num_subcores=NS)

    @functools.partial(
        pl.kernel,
        out_type=(
            jax.ShapeDtypeStruct((max_len, n_cls, d), jnp.float32),
            jax.ShapeDtypeStruct((max_len, MPAD), jnp.int32),
        ),
        mesh=mesh,
        compiler_params=pltpu.CompilerParams(needs_layout_passes=False),
        scratch_types=[
            pltpu.VMEM((2, W, W, d), jnp.float32),
            pltpu.VMEM((3, W, d), jnp.float32),
            pltpu.VMEM((d,), jnp.float32),
            pltpu.VMEM((d,), jnp.float32),
            pltpu.VMEM((CPW,), jnp.int32),
            pltpu.VMEM((MW,), jnp.int32),
            pltpu.VMEM((max_len, MW), jnp.int32),
            pltpu.VMEM_SHARED((hdr, W, d), jnp.float32),
            pltpu.SemaphoreType.DMA((2,)),
            pltpu.SemaphoreType.DMA((3,)),
            pltpu.SemaphoreType.DMA,
        ],
    )
    def sc_kernel(ctx_h, ce_h, clsv_h, sep_h, lens_h, out_h, mask_h,
                  gbuf, ab, cls_v, sep_v, lens_v, lens_m, mbuf,
                  hrep, semG, semT, semH):
        c = lax.axis_index("c")
        s = lax.axis_index("s")
        w = s * NC + c
        base = pl.multiple_of(w * CPW, W)
        full = base + CPW <= n_cls
        ng = jnp.where(full, G, rem // W)

        pltpu.sync_copy(clsv_h, cls_v)
        pltpu.sync_copy(sep_h, sep_v)

        @pl.when(full)
        def _stage_lens_full():
            pltpu.sync_copy(lens_h.at[pl.ds(base, CPW)], lens_v)

        @pl.when(jnp.logical_not(full))
        def _stage_lens_rem():
            pltpu.sync_copy(lens_h.at[pl.ds(base, rem)], lens_v.at[pl.ds(0, rem)])

        iot = lax.iota(jnp.int32, 16)
        nch = d // 16

        def fill_hrep(p):
            pm1 = jnp.maximum(p - 1, 0)

            @pl.when(p > 0)
            def _stage_ctx_tile():
                pltpu.sync_copy(
                    ctx_h.at[pl.ds(pl.multiple_of((pm1 // W) * W, W), W)],
                    ab.at[1])

            r_in_tile = lax.rem(pm1, W)

            def fchunk(cc, carry):
                o = cc * 16
                v = jnp.where(p == 0, cls_v[pl.ds(o, 16)],
                              ab[1, r_in_tile, pl.ds(o, 16)])
                for r in range(W):
                    ab[0, r, pl.ds(o, 16)] = v
                return carry

            lax.fori_loop(0, nch, fchunk, 0)
            pltpu.sync_copy(ab.at[0], hrep.at[p])

        fill_hrep(s)

        @pl.when(s == 0)
        def _fill_last():
            fill_hrep(jnp.int32(hdr - 1))

        plsc.subcore_barrier()

        def lenof(j):
            lane = jnp.bitwise_and(j, 15)
            lv = lens_v[pl.ds(j - lane, 16)]
            return jnp.sum(jnp.where(iot == lane, lv, 0))

        def stage(g):
            pltpu.make_async_copy(
                ce_h.at[pl.ds(base + g * W, W)], gbuf.at[lax.rem(g, 2)],
                semG.at[lax.rem(g, 2)]).start()

        stage(0)

        def body(g, carry):
            gslot = lax.rem(g, 2)
            gb = pl.multiple_of(base + g * W, W)

            @pl.when(g + 1 < ng)
            def _next():
                stage(g + 1)

            def hout(p, carry2):
                pltpu.make_async_copy(
                    hrep.at[p], out_h.at[p, pl.ds(gb, W)], semH).start()
                return carry2

            lax.fori_loop(0, hdr, hout, 0)

            pltpu.make_async_copy(
                ce_h.at[pl.ds(gb, W)], gbuf.at[gslot], semG.at[gslot]).wait()

            Ls = [lenof(g * W + i) for i in range(W)]

            for k in range(T):
                a = g * T + k
                aslot = lax.rem(a, 3)

                @pl.when(a >= 3)
                def _freeslot():
                    pltpu.make_async_copy(
                        ab.at[aslot], out_h.at[hdr, pl.ds(gb, W)],
                        semT.at[aslot]).wait()

                zv = jnp.zeros((16,), jnp.float32)
                for i in range(W):
                    Li = Ls[i]

                    def _sepz(i=i, k=k, aslot=aslot, Li=Li):
                        def sz(cc, carry2):
                            o = cc * 16
                            ab[aslot, i, pl.ds(o, 16)] = jnp.where(
                                Li == k, sep_v[pl.ds(o, 16)], zv)
                            return carry2
                        lax.fori_loop(0, nch, sz, 0)

                    if k < W:
                        def _copy(i=i, k=k, aslot=aslot):
                            def cp(cc, carry2):
                                o = cc * 16
                                ab[aslot, i, pl.ds(o, 16)] = \
                                    gbuf[gslot, i, k, pl.ds(o, 16)]
                                return carry2
                            lax.fori_loop(0, nch, cp, 0)

                        pl.when(Li > k)(_copy)
                        pl.when(Li <= k)(_sepz)
                    else:
                        _sepz()
                pltpu.make_async_copy(
                    ab.at[aslot], out_h.at[hdr + k, pl.ds(gb, W)],
                    semT.at[aslot]).start()
            return carry

        lax.fori_loop(0, ng, body, 0)

        mfull_w = jnp.logical_and(lax.rem(w, 4) == 0, base + MW <= n_cls)
        medge_w = jnp.logical_and(lax.rem(w, 4) == 0, base + MW > n_cls)
        mbase = pl.multiple_of((w // 4) * MW, MW)

        def build_mask():
            for pc in range(MW // 16):
                lv = lens_m[pl.ds(pc * 16, 16)]
                for p in range(max_len):
                    mbuf[p, pl.ds(pc * 16, 16)] = jnp.where(
                        lv > p - (2 + n_ctx), 1, 0).astype(jnp.int32)

        @pl.when(mfull_w)
        def _mask_full():
            pltpu.sync_copy(lens_h.at[pl.ds(mbase, MW)], lens_m)
            build_mask()
            pltpu.sync_copy(mbuf, mask_h.at[pl.ds(0, max_len), pl.ds(mbase, MW)])

        @pl.when(medge_w)
        def _mask_edge():
            pltpu.sync_copy(lens_h.at[pl.ds(mbase, mrem)],
                            lens_m.at[pl.ds(0, mrem)])
            build_mask()
            pltpu.sync_copy(mbuf, mask_h.at[pl.ds(0, max_len), pl.ds(mbase, MW)])

        for r in range(3):
            pltpu.make_async_copy(
                ab.at[r], out_h.at[hdr, pl.ds(base, W)], semT.at[r]).wait()

        def drain_hdr(j, carry):
            pltpu.make_async_copy(
                out_h.at[0, pl.ds(base, W)], hrep.at[0], semH).wait()
            return carry

        lax.fori_loop(0, hdr * ng, drain_hdr, 0)

    ft, mt = sc_kernel(ctx, class_embeds, cls_embed, sep_embed, class_lens)
    return jnp.swapaxes(ft, 0, 1), jnp.swapaxes(mt[:, :n_cls], 0, 1)


# speedup vs baseline: 1.5874x; 1.5874x over previous
import functools

import jax
import jax.numpy as jnp
from jax import lax
from jax.experimental import pallas as pl
from jax.experimental.pallas import tpu as pltpu
from jax.experimental.pallas import tpu_sc as plsc


def kernel(ctx, class_embeds, cls_embed, sep_embed, class_lens):
    n_cls, W, d = class_embeds.shape
    n_ctx = ctx.shape[0]
    hdr = 1 + n_ctx
    max_len = 2 + n_ctx + W
    T = W + 1

    NC, NS = 2, 16
    NW = NC * NS
    CPW = -(-n_cls // NW)
    rem = n_cls - (NW - 1) * CPW
    G = CPW // W
    MW = 128
    MPAD = -(-n_cls // MW) * MW
    mrem = n_cls % MW or MW

    mesh = plsc.VectorSubcoreMesh(
        core_axis_name="c", subcore_axis_name="s",
        num_cores=NC, num_subcores=NS)

    @functools.partial(
        pl.kernel,
        out_type=(
            jax.ShapeDtypeStruct((max_len, n_cls, d), jnp.float32),
            jax.ShapeDtypeStruct((max_len, MPAD), jnp.int32),
        ),
        mesh=mesh,
        compiler_params=pltpu.CompilerParams(needs_layout_passes=False),
        scratch_types=[
            pltpu.VMEM((2, W, W, d), jnp.float32),
            pltpu.VMEM((3, W, d), jnp.float32),
            pltpu.VMEM((d,), jnp.float32),
            pltpu.VMEM((d,), jnp.float32),
            pltpu.VMEM((CPW,), jnp.int32),
            pltpu.VMEM((MW,), jnp.int32),
            pltpu.VMEM((max_len, MW), jnp.int32),
            pltpu.VMEM_SHARED((hdr, W, d), jnp.float32),
            pltpu.SemaphoreType.DMA((2,)),
            pltpu.SemaphoreType.DMA((3,)),
            pltpu.SemaphoreType.DMA,
        ],
    )
    def sc_kernel(ctx_h, ce_h, clsv_h, sep_h, lens_h, out_h, mask_h,
                  gbuf, ab, cls_v, sep_v, lens_v, lens_m, mbuf,
                  hrep, semG, semT, semH):
        c = lax.axis_index("c")
        s = lax.axis_index("s")
        w = s * NC + c
        base = pl.multiple_of(w * CPW, W)
        full = base + CPW <= n_cls
        ng = jnp.where(full, G, rem // W)

        pltpu.sync_copy(clsv_h, cls_v)
        pltpu.sync_copy(sep_h, sep_v)

        @pl.when(full)
        def _stage_lens_full():
            pltpu.sync_copy(lens_h.at[pl.ds(base, CPW)], lens_v)

        @pl.when(jnp.logical_not(full))
        def _stage_lens_rem():
            pltpu.sync_copy(lens_h.at[pl.ds(base, rem)], lens_v.at[pl.ds(0, rem)])

        iot = lax.iota(jnp.int32, 16)
        nch = d // 16

        def fill_hrep(p):
            pm1 = jnp.maximum(p - 1, 0)

            @pl.when(p > 0)
            def _stage_ctx_tile():
                pltpu.sync_copy(
                    ctx_h.at[pl.ds(pl.multiple_of((pm1 // W) * W, W), W)],
                    ab.at[1])

            r_in_tile = lax.rem(pm1, W)

            def fchunk(cc, carry):
                o = cc * 16
                v = jnp.where(p == 0, cls_v[pl.ds(o, 16)],
                              ab[1, r_in_tile, pl.ds(o, 16)])
                for r in range(W):
                    ab[0, r, pl.ds(o, 16)] = v
                return carry

            lax.fori_loop(0, nch, fchunk, 0)
            pltpu.sync_copy(ab.at[0], hrep.at[p])

        fill_hrep(s)

        @pl.when(s == 0)
        def _fill_last():
            fill_hrep(jnp.int32(hdr - 1))

        plsc.subcore_barrier()

        def lenof(j):
            lane = jnp.bitwise_and(j, 15)
            lv = lens_v[pl.ds(j - lane, 16)]
            return jnp.sum(jnp.where(iot == lane, lv, 0))

        def stage(g):
            pltpu.make_async_copy(
                ce_h.at[pl.ds(base + g * W, W)], gbuf.at[lax.rem(g, 2)],
                semG.at[lax.rem(g, 2)]).start()

        stage(0)

        def body(g, carry):
            gslot = lax.rem(g, 2)
            gb = pl.multiple_of(base + g * W, W)

            @pl.when(g + 1 < ng)
            def _next():
                stage(g + 1)

            def hout(p, carry2):
                pltpu.make_async_copy(
                    hrep.at[p], out_h.at[p, pl.ds(gb, W)], semH).start()
                return carry2

            lax.fori_loop(0, hdr, hout, 0)

            pltpu.make_async_copy(
                ce_h.at[pl.ds(gb, W)], gbuf.at[gslot], semG.at[gslot]).wait()

            Ls = [lenof(g * W + i) for i in range(W)]

            for k in range(T):
                a = g * T + k
                aslot = lax.rem(a, 3)

                @pl.when(a >= 3)
                def _freeslot():
                    pltpu.make_async_copy(
                        ab.at[aslot], out_h.at[hdr, pl.ds(gb, W)],
                        semT.at[aslot]).wait()

                @functools.partial(plsc.parallel_loop, 0, nch, unroll=2)
                def achunk(cc):
                    o = cc * 16
                    sv = sep_v[pl.ds(o, 16)]
                    zv = jnp.zeros((16,), jnp.float32)
                    for i in range(W):
                        if k < W:
                            val = jnp.where(
                                Ls[i] > k, gbuf[gslot, i, k, pl.ds(o, 16)],
                                jnp.where(Ls[i] == k, sv, zv))
                        else:
                            val = jnp.where(Ls[i] == k, sv, zv)
                        ab[aslot, i, pl.ds(o, 16)] = val
                pltpu.make_async_copy(
                    ab.at[aslot], out_h.at[hdr + k, pl.ds(gb, W)],
                    semT.at[aslot]).start()
            return carry

        lax.fori_loop(0, ng, body, 0)

        mfull_w = jnp.logical_and(lax.rem(w, 4) == 0, base + MW <= n_cls)
        medge_w = jnp.logical_and(lax.rem(w, 4) == 0, base + MW > n_cls)
        mbase = pl.multiple_of((w // 4) * MW, MW)

        def build_mask():
            for pc in range(MW // 16):
                lv = lens_m[pl.ds(pc * 16, 16)]
                for p in range(max_len):
                    mbuf[p, pl.ds(pc * 16, 16)] = jnp.where(
                        lv > p - (2 + n_ctx), 1, 0).astype(jnp.int32)

        @pl.when(mfull_w)
        def _mask_full():
            pltpu.sync_copy(lens_h.at[pl.ds(mbase, MW)], lens_m)
            build_mask()
            pltpu.sync_copy(mbuf, mask_h.at[pl.ds(0, max_len), pl.ds(mbase, MW)])

        @pl.when(medge_w)
        def _mask_edge():
            pltpu.sync_copy(lens_h.at[pl.ds(mbase, mrem)],
                            lens_m.at[pl.ds(0, mrem)])
            build_mask()
            pltpu.sync_copy(mbuf, mask_h.at[pl.ds(0, max_len), pl.ds(mbase, MW)])

        for r in range(3):
            pltpu.make_async_copy(
                ab.at[r], out_h.at[hdr, pl.ds(base, W)], semT.at[r]).wait()

        def drain_hdr(j, carry):
            pltpu.make_async_copy(
                out_h.at[0, pl.ds(base, W)], hrep.at[0], semH).wait()
            return carry

        lax.fori_loop(0, hdr * ng, drain_hdr, 0)

    ft, mt = sc_kernel(ctx, class_embeds, cls_embed, sep_embed, class_lens)
    return jnp.swapaxes(ft, 0, 1), jnp.swapaxes(mt[:, :n_cls], 0, 1)
